# baseline (speedup 1.0000x reference)
import jax
import jax.numpy as jnp
from jax.experimental import pallas as pl

_ALPHA = 0.05
_KHOPS = 2
_KEIG = 16
_GB = 128


def _mixer_block(x_ref, adj_ref, pe_ref, we_ref, be_ref, W0_ref, b0_ref,
                 W1_ref, b1_ref, g0_ref, bt0_ref, g1_ref, bt1_ref, out_ref):
    x = x_ref[...]
    adj = adj_ref[...]
    pe = pe_ref[...]
    we = we_ref[0, 0]
    be = be_ref[0, 0]
    _, N, _ = x.shape

    mask = (adj > 0).astype(jnp.float32)
    w_t = jax.nn.softplus(adj * we + be) * mask
    eye = jnp.eye(N, dtype=jnp.float32)
    diag_present = jnp.sum(mask * eye[None], axis=-1)
    w_t = w_t + (1.0 - diag_present)[..., None] * eye[None]
    deg = jnp.sum(w_t, axis=-2)
    dinv = jnp.where(deg > 0, jax.lax.rsqrt(deg), 0.0)
    anorm = dinv[:, :, None] * w_t * dinv[:, None, :]

    s = jnp.sum(pe, axis=-2, keepdims=True)
    signs = jnp.sign(s)
    signs = jnp.where(signs == 0, jnp.ones_like(signs), signs)
    pe = pe * signs
    pe = jnp.nan_to_num(pe, nan=0.0, posinf=1.0, neginf=-1.0)
    xpe = jnp.concatenate([x, pe], axis=-1)

    def prop(v):
        return jax.lax.dot_general(anorm, v, (((1,), (1,)), ((0,), (0,))),
                                   preferred_element_type=jnp.float32)

    def ssg(v, W, b):
        h = _ALPHA * v
        vk = v
        for _ in range(_KHOPS):
            vk = prop(vk)
            h = h + (1.0 - _ALPHA) / _KHOPS * vk
        y = jax.lax.dot_general(h, W, (((2,), (1,)), ((), ())),
                                preferred_element_type=jnp.float32)
        return y + b[...][None]

    def ln(v, g, b):
        m = jnp.mean(v, axis=-1, keepdims=True)
        c = v - m
        var = jnp.mean(c * c, axis=-1, keepdims=True)
        return c * jax.lax.rsqrt(var + 1e-5) * g[...][None] + b[...][None]

    h0 = ln(ssg(xpe, W0_ref[...], b0_ref), g0_ref, bt0_ref)
    h1 = ln(ssg(h0, W1_ref[...], b1_ref), g1_ref, bt1_ref)
    out_ref[...] = h0 + h1


def kernel(features, adjacency, we, be, W0, b0, W1, b1, g0, bt0, g1, bt1):
    B, N, T, DM = features.shape
    G = B * T
    x = jnp.transpose(features, (0, 2, 1, 3)).reshape(G, N, DM)
    adj = adjacency.reshape(G, N, N)

    degrees = jnp.clip(jnp.sum(adj, axis=-1), 1e-6, None)
    dpe = 1.0 / jnp.sqrt(degrees)
    eye = jnp.eye(N, dtype=jnp.float32)
    lap = eye[None] - dpe[:, :, None] * adj * dpe[:, None, :] + 1e-5 * eye[None]
    _, vecs = jnp.linalg.eigh(lap)
    pe = vecs[..., :_KEIG]

    be2 = be.reshape(1, 1)
    params1d = [p.reshape(1, -1) for p in (b0, b1, g0, bt0, g1, bt1)]
    b0r, b1r, g0r, bt0r, g1r, bt1r = params1d

    grid = G // _GB
    full = lambda shape: pl.BlockSpec(shape, lambda i: (0,) * len(shape))
    out = pl.pallas_call(
        _mixer_block,
        grid=(grid,),
        in_specs=[
            pl.BlockSpec((_GB, N, DM), lambda i: (i, 0, 0)),
            pl.BlockSpec((_GB, N, N), lambda i: (i, 0, 0)),
            pl.BlockSpec((_GB, N, _KEIG), lambda i: (i, 0, 0)),
            full((1, 1)),
            full((1, 1)),
            full(W0.shape),
            full((1, DM)),
            full(W1.shape),
            full((1, DM)),
            full((1, DM)),
            full((1, DM)),
            full((1, DM)),
            full((1, DM)),
        ],
        out_specs=pl.BlockSpec((_GB, N, DM), lambda i: (i, 0, 0)),
        out_shape=jax.ShapeDtypeStruct((G, N, DM), jnp.float32),
    )(x, adj, pe, we, be2, W0, b0r, W1, b1r, g0r, bt0r, g1r, bt1r)

    return jnp.transpose(out.reshape(B, T, N, DM), (0, 2, 1, 3))


# speedup vs baseline: 7.3343x; 7.3343x over previous
import jax
import jax.numpy as jnp
import numpy as np
from jax.experimental import pallas as pl

_ALPHA = 0.05
_KHOPS = 2
_KEIG = 16
_GB = 128
_GE = 512
_NP = 20
_NSWEEPS = 10
_BIG = 1.0e6


def _round_robin_rounds(n):
    others = list(range(1, n))
    rounds = []
    for r in range(n - 1):
        rot = others[r:] + others[:r]
        line = [0] + rot
        partner = [0] * n
        for i in range(n // 2):
            a, b = line[i], line[n - 1 - i]
            partner[a] = b
            partner[b] = a
        rounds.append(tuple(partner))
    return rounds


_ROUNDS = _round_robin_rounds(_NP)


def _perm0(x, perm):
    return jnp.concatenate([x[p:p + 1] for p in perm], axis=0)


def _perm1(x, perm):
    return jnp.concatenate([x[:, p:p + 1, :] for p in perm], axis=1)


def _iota_eye(n):
    ri = jax.lax.broadcasted_iota(jnp.int32, (n, n), 0)
    ci = jax.lax.broadcasted_iota(jnp.int32, (n, n), 1)
    return (ri == ci).astype(jnp.float32)


def _eig_block(pm_ref, adj_ref, adjt_ref, pe_ref):
    adj = adj_ref[...]
    adjt = adjt_ref[...]
    N, _, Ge = adj.shape

    deg = jnp.clip(jnp.sum(adj, axis=1), 1e-6, None)
    dpe = jax.lax.rsqrt(deg)
    eye = _iota_eye(N)
    m = dpe[:, None, :] * adj * dpe[None, :, :]
    mt = dpe[None, :, :] * adjt * dpe[:, None, :]
    lap = eye[:, :, None] * (1.0 + 1e-5) - 0.5 * (m + mt)

    zc = jnp.zeros((N, 1, Ge), dtype=jnp.float32)
    top = jnp.concatenate([lap, zc], axis=1)
    bot = jnp.concatenate(
        [jnp.zeros((1, N, Ge), dtype=jnp.float32),
         jnp.full((1, 1, Ge), _BIG, dtype=jnp.float32)], axis=1)
    A = jnp.concatenate([top, bot], axis=0)

    eyep = _iota_eye(_NP)
    V = eyep[:, :, None] + jnp.zeros((1, 1, Ge), dtype=jnp.float32)
    eyem = eyep[:, :, None]

    def sweep(_, carry):
        A, V = carry
        for r, partner in enumerate(_ROUNDS):
            pmj = pm_ref[r][:, :, None]
            diag = jnp.sum(A * eyem, axis=1)
            apq = jnp.sum(A * jnp.abs(pmj), axis=1)
            ts = jnp.sum(pmj, axis=1)
            dq = _perm0(diag, partner)
            apq_safe = jnp.where(apq == 0.0, 1.0, apq)
            tau = (dq - diag) / (2.0 * apq_safe)
            sgn = jnp.where(tau > 0.0, 1.0, jnp.where(tau < 0.0, -1.0, ts))
            t = sgn / (jnp.abs(tau) + jnp.sqrt(1.0 + tau * tau))
            t = jnp.where(apq == 0.0, 0.0, t)
            c = jax.lax.rsqrt(1.0 + t * t)
            s = t * c
            S = -s
            B = c[:, None, :] * A + S[:, None, :] * _perm0(A, partner)
            A = c[None] * B + S[None] * _perm1(B, partner)
            V = c[None] * V + S[None] * _perm1(V, partner)
        return A, V

    A, V = jax.lax.fori_loop(0, _NSWEEPS, sweep, (A, V))

    ev = jnp.sum(A * eyem, axis=1)
    e1 = ev[:, None, :]
    e2 = ev[None, :, :]
    ri = jax.lax.broadcasted_iota(jnp.int32, (_NP, _NP), 0)
    ci = jax.lax.broadcasted_iota(jnp.int32, (_NP, _NP), 1)
    tie = (ri < ci).astype(jnp.float32)
    cnt = (e1 < e2).astype(jnp.float32) + (e1 == e2).astype(jnp.float32) * tie[:, :, None]
    rank = jnp.sum(cnt, axis=0)

    cols = []
    for k in range(_KEIG):
        sel = (rank == float(k)).astype(jnp.float32)
        col = jnp.sum(V * sel[None], axis=1)
        ssum = jnp.sum(col[:N], axis=0, keepdims=True)
        sg = jnp.sign(ssum)
        sg = jnp.where(sg == 0.0, 1.0, sg)
        col = col * sg
        col = jnp.nan_to_num(col, nan=0.0, posinf=1.0, neginf=-1.0)
        cols.append(col[:N, None, :])
    pe_ref[...] = jnp.concatenate(cols, axis=1)


def _mixer_block(x_ref, adj_ref, pe_ref, we_ref, be_ref, W0_ref, b0_ref,
                 W1_ref, b1_ref, g0_ref, bt0_ref, g1_ref, bt1_ref, out_ref):
    x = x_ref[...]
    adj = adj_ref[...]
    pe = pe_ref[...]
    we = we_ref[0, 0]
    be = be_ref[0, 0]
    _, N, _ = x.shape

    mask = (adj > 0).astype(jnp.float32)
    w_t = jax.nn.softplus(adj * we + be) * mask
    eye = jnp.eye(N, dtype=jnp.float32)
    diag_present = jnp.sum(mask * eye[None], axis=-1)
    w_t = w_t + (1.0 - diag_present)[..., None] * eye[None]
    deg = jnp.sum(w_t, axis=-2)
    dinv = jnp.where(deg > 0, jax.lax.rsqrt(deg), 0.0)
    anorm = dinv[:, :, None] * w_t * dinv[:, None, :]

    xpe = jnp.concatenate([x, pe], axis=-1)

    def prop(v):
        return jax.lax.dot_general(anorm, v, (((1,), (1,)), ((0,), (0,))),
                                   preferred_element_type=jnp.float32)

    def ssg(v, W, b):
        h = _ALPHA * v
        vk = v
        for _ in range(_KHOPS):
            vk = prop(vk)
            h = h + (1.0 - _ALPHA) / _KHOPS * vk
        y = jax.lax.dot_general(h, W, (((2,), (1,)), ((), ())),
                                preferred_element_type=jnp.float32)
        return y + b[...][None]

    def ln(v, g, b):
        m = jnp.mean(v, axis=-1, keepdims=True)
        c = v - m
        var = jnp.mean(c * c, axis=-1, keepdims=True)
        return c * jax.lax.rsqrt(var + 1e-5) * g[...][None] + b[...][None]

    h0 = ln(ssg(xpe, W0_ref[...], b0_ref), g0_ref, bt0_ref)
    h1 = ln(ssg(h0, W1_ref[...], b1_ref), g1_ref, bt1_ref)
    out_ref[...] = h0 + h1


def kernel(features, adjacency, we, be, W0, b0, W1, b1, g0, bt0, g1, bt1):
    B, N, T, DM = features.shape
    G = B * T
    x = jnp.transpose(features, (0, 2, 1, 3)).reshape(G, N, DM)
    adj = adjacency.reshape(G, N, N)

    pm_all = np.zeros((len(_ROUNDS), _NP, _NP), dtype=np.float32)
    for r, partner in enumerate(_ROUNDS):
        for i in range(_NP):
            pm_all[r, i, partner[i]] = 1.0 if i < partner[i] else -1.0
    adj_t = jnp.transpose(adj, (1, 2, 0))
    adjt_t = jnp.transpose(adj, (2, 1, 0))
    pe_t = pl.pallas_call(
        _eig_block,
        grid=(G // _GE,),
        in_specs=[
            pl.BlockSpec((len(_ROUNDS), _NP, _NP), lambda i: (0, 0, 0)),
            pl.BlockSpec((N, N, _GE), lambda i: (0, 0, i)),
            pl.BlockSpec((N, N, _GE), lambda i: (0, 0, i)),
        ],
        out_specs=pl.BlockSpec((N, _KEIG, _GE), lambda i: (0, 0, i)),
        out_shape=jax.ShapeDtypeStruct((N, _KEIG, G), jnp.float32),
    )(jnp.asarray(pm_all), adj_t, adjt_t)
    pe = jnp.transpose(pe_t, (2, 0, 1))

    be2 = be.reshape(1, 1)
    b0r, b1r, g0r, bt0r, g1r, bt1r = (
        p.reshape(1, -1) for p in (b0, b1, g0, bt0, g1, bt1))

    full = lambda shape: pl.BlockSpec(shape, lambda i: (0,) * len(shape))
    out = pl.pallas_call(
        _mixer_block,
        grid=(G // _GB,),
        in_specs=[
            pl.BlockSpec((_GB, N, DM), lambda i: (i, 0, 0)),
            pl.BlockSpec((_GB, N, N), lambda i: (i, 0, 0)),
            pl.BlockSpec((_GB, N, _KEIG), lambda i: (i, 0, 0)),
            full((1, 1)),
            full((1, 1)),
            full(W0.shape),
            full((1, DM)),
            full(W1.shape),
            full((1, DM)),
            full((1, DM)),
            full((1, DM)),
            full((1, DM)),
            full((1, DM)),
        ],
        out_specs=pl.BlockSpec((_GB, N, DM), lambda i: (i, 0, 0)),
        out_shape=jax.ShapeDtypeStruct((G, N, DM), jnp.float32),
    )(x, adj, pe, we, be2, W0, b0r, W1, b1r, g0r, bt0r, g1r, bt1r)

    return jnp.transpose(out.reshape(B, T, N, DM), (0, 2, 1, 3))


# speedup vs baseline: 10.2068x; 1.3917x over previous
import jax
import jax.numpy as jnp
import numpy as np
from jax.experimental import pallas as pl

_ALPHA = 0.05
_KHOPS = 2
_KEIG = 16
_GB = 128
_GE = 512
_NP = 20
_NSWEEPS = 7
_BIG = 1.0e6


def _round_robin_rounds(n):
    others = list(range(1, n))
    rounds = []
    for r in range(n - 1):
        rot = others[r:] + others[:r]
        line = [0] + rot
        partner = [0] * n
        for i in range(n // 2):
            a, b = line[i], line[n - 1 - i]
            partner[a] = b
            partner[b] = a
        rounds.append(tuple(partner))
    return rounds


_ROUNDS = _round_robin_rounds(_NP)


def _perm0(x, perm):
    return jnp.concatenate([x[p:p + 1] for p in perm], axis=0)


def _perm1(x, perm):
    return jnp.concatenate([x[:, p:p + 1, :] for p in perm], axis=1)


def _iota_eye(n):
    ri = jax.lax.broadcasted_iota(jnp.int32, (n, n), 0)
    ci = jax.lax.broadcasted_iota(jnp.int32, (n, n), 1)
    return (ri == ci).astype(jnp.float32)


def _eig_block(pm_ref, adj_ref, adjt_ref, pe_ref):
    adj = adj_ref[...]
    adjt = adjt_ref[...]
    N, _, Ge = adj.shape

    deg = jnp.clip(jnp.sum(adj, axis=1), 1e-6, None)
    dpe = jax.lax.rsqrt(deg)
    eye = _iota_eye(N)
    m = dpe[:, None, :] * adj * dpe[None, :, :]
    mt = dpe[None, :, :] * adjt * dpe[:, None, :]
    lap = eye[:, :, None] * (1.0 + 1e-5) - 0.5 * (m + mt)

    zc = jnp.zeros((N, 1, Ge), dtype=jnp.float32)
    top = jnp.concatenate([lap, zc], axis=1)
    bot = jnp.concatenate(
        [jnp.zeros((1, N, Ge), dtype=jnp.float32),
         jnp.full((1, 1, Ge), _BIG, dtype=jnp.float32)], axis=1)
    A = jnp.concatenate([top, bot], axis=0)

    eyep = _iota_eye(_NP)
    V = eyep[:, :, None] + jnp.zeros((1, 1, Ge), dtype=jnp.float32)
    eyem = eyep[:, :, None]

    def sweep(_, carry):
        A, V = carry
        for r, partner in enumerate(_ROUNDS):
            pmj = pm_ref[r][:, :, None]
            diag = jnp.sum(A * eyem, axis=1)
            apq = jnp.sum(A * jnp.abs(pmj), axis=1)
            ts = jnp.sum(pmj, axis=1)
            dq = _perm0(diag, partner)
            apq_safe = jnp.where(apq == 0.0, 1.0, apq)
            tau = (dq - diag) / (2.0 * apq_safe)
            sgn = jnp.where(tau > 0.0, 1.0, jnp.where(tau < 0.0, -1.0, ts))
            t = sgn / (jnp.abs(tau) + jnp.sqrt(1.0 + tau * tau))
            t = jnp.where(apq == 0.0, 0.0, t)
            c = jax.lax.rsqrt(1.0 + t * t)
            s = t * c
            S = -s
            B = c[:, None, :] * A + S[:, None, :] * _perm0(A, partner)
            A = c[None] * B + S[None] * _perm1(B, partner)
            V = c[None] * V + S[None] * _perm1(V, partner)
        return A, V

    A, V = jax.lax.fori_loop(0, _NSWEEPS, sweep, (A, V))

    ev = jnp.sum(A * eyem, axis=1)
    e1 = ev[:, None, :]
    e2 = ev[None, :, :]
    ri = jax.lax.broadcasted_iota(jnp.int32, (_NP, _NP), 0)
    ci = jax.lax.broadcasted_iota(jnp.int32, (_NP, _NP), 1)
    tie = (ri < ci).astype(jnp.float32)
    cnt = (e1 < e2).astype(jnp.float32) + (e1 == e2).astype(jnp.float32) * tie[:, :, None]
    rank = jnp.sum(cnt, axis=0)

    cols = []
    for k in range(_KEIG):
        sel = (rank == float(k)).astype(jnp.float32)
        col = jnp.sum(V * sel[None], axis=1)
        ssum = jnp.sum(col[:N], axis=0, keepdims=True)
        sg = jnp.sign(ssum)
        sg = jnp.where(sg == 0.0, 1.0, sg)
        col = col * sg
        col = jnp.nan_to_num(col, nan=0.0, posinf=1.0, neginf=-1.0)
        cols.append(col[:N, None, :])
    pe_ref[...] = jnp.concatenate(cols, axis=1)


def _mixer_block(x_ref, adj_ref, pe_ref, we_ref, be_ref, W0_ref, b0_ref,
                 W1_ref, b1_ref, g0_ref, bt0_ref, g1_ref, bt1_ref, out_ref):
    x = x_ref[...]
    adj = adj_ref[...]
    pe = pe_ref[...]
    we = we_ref[0, 0]
    be = be_ref[0, 0]
    _, N, _ = x.shape

    mask = (adj > 0).astype(jnp.float32)
    w_t = jax.nn.softplus(adj * we + be) * mask
    eye = jnp.eye(N, dtype=jnp.float32)
    diag_present = jnp.sum(mask * eye[None], axis=-1)
    w_t = w_t + (1.0 - diag_present)[..., None] * eye[None]
    deg = jnp.sum(w_t, axis=-2)
    dinv = jnp.where(deg > 0, jax.lax.rsqrt(deg), 0.0)
    anorm = dinv[:, :, None] * w_t * dinv[:, None, :]

    xpe = jnp.concatenate([x, pe], axis=-1)

    def prop(v):
        return jax.lax.dot_general(anorm, v, (((1,), (1,)), ((0,), (0,))),
                                   preferred_element_type=jnp.float32)

    def ssg(v, W, b):
        h = _ALPHA * v
        vk = v
        for _ in range(_KHOPS):
            vk = prop(vk)
            h = h + (1.0 - _ALPHA) / _KHOPS * vk
        y = jax.lax.dot_general(h, W, (((2,), (1,)), ((), ())),
                                preferred_element_type=jnp.float32)
        return y + b[...][None]

    def ln(v, g, b):
        m = jnp.mean(v, axis=-1, keepdims=True)
        c = v - m
        var = jnp.mean(c * c, axis=-1, keepdims=True)
        return c * jax.lax.rsqrt(var + 1e-5) * g[...][None] + b[...][None]

    h0 = ln(ssg(xpe, W0_ref[...], b0_ref), g0_ref, bt0_ref)
    h1 = ln(ssg(h0, W1_ref[...], b1_ref), g1_ref, bt1_ref)
    out_ref[...] = h0 + h1


def kernel(features, adjacency, we, be, W0, b0, W1, b1, g0, bt0, g1, bt1):
    B, N, T, DM = features.shape
    G = B * T
    x = jnp.transpose(features, (0, 2, 1, 3)).reshape(G, N, DM)
    adj = adjacency.reshape(G, N, N)

    pm_all = np.zeros((len(_ROUNDS), _NP, _NP), dtype=np.float32)
    for r, partner in enumerate(_ROUNDS):
        for i in range(_NP):
            pm_all[r, i, partner[i]] = 1.0 if i < partner[i] else -1.0
    adj_t = jnp.transpose(adj, (1, 2, 0))
    adjt_t = jnp.transpose(adj, (2, 1, 0))
    pe_t = pl.pallas_call(
        _eig_block,
        grid=(G // _GE,),
        in_specs=[
            pl.BlockSpec((len(_ROUNDS), _NP, _NP), lambda i: (0, 0, 0)),
            pl.BlockSpec((N, N, _GE), lambda i: (0, 0, i)),
            pl.BlockSpec((N, N, _GE), lambda i: (0, 0, i)),
        ],
        out_specs=pl.BlockSpec((N, _KEIG, _GE), lambda i: (0, 0, i)),
        out_shape=jax.ShapeDtypeStruct((N, _KEIG, G), jnp.float32),
    )(jnp.asarray(pm_all), adj_t, adjt_t)
    pe = jnp.transpose(pe_t, (2, 0, 1))

    be2 = be.reshape(1, 1)
    b0r, b1r, g0r, bt0r, g1r, bt1r = (
        p.reshape(1, -1) for p in (b0, b1, g0, bt0, g1, bt1))

    full = lambda shape: pl.BlockSpec(shape, lambda i: (0,) * len(shape))
    out = pl.pallas_call(
        _mixer_block,
        grid=(G // _GB,),
        in_specs=[
            pl.BlockSpec((_GB, N, DM), lambda i: (i, 0, 0)),
            pl.BlockSpec((_GB, N, N), lambda i: (i, 0, 0)),
            pl.BlockSpec((_GB, N, _KEIG), lambda i: (i, 0, 0)),
            full((1, 1)),
            full((1, 1)),
            full(W0.shape),
            full((1, DM)),
            full(W1.shape),
            full((1, DM)),
            full((1, DM)),
            full((1, DM)),
            full((1, DM)),
            full((1, DM)),
        ],
        out_specs=pl.BlockSpec((_GB, N, DM), lambda i: (i, 0, 0)),
        out_shape=jax.ShapeDtypeStruct((G, N, DM), jnp.float32),
    )(x, adj, pe, we, be2, W0, b0r, W1, b1r, g0r, bt0r, g1r, bt1r)

    return jnp.transpose(out.reshape(B, T, N, DM), (0, 2, 1, 3))


# speedup vs baseline: 11.2676x; 1.1039x over previous
import jax
import jax.numpy as jnp
import numpy as np
from jax.experimental import pallas as pl

_ALPHA = 0.05
_KHOPS = 2
_KEIG = 16
_GB = 128
_GE = 512
_NP = 20
_NSWEEPS = 7
_BIG = 1.0e6


def _round_robin_rounds(n):
    others = list(range(1, n))
    rounds = []
    for r in range(n - 1):
        rot = others[r:] + others[:r]
        line = [0] + rot
        partner = [0] * n
        for i in range(n // 2):
            a, b = line[i], line[n - 1 - i]
            partner[a] = b
            partner[b] = a
        rounds.append(tuple(partner))
    return rounds


_ROUNDS = _round_robin_rounds(_NP)


def _perm0(x, perm):
    return jnp.concatenate([x[p:p + 1] for p in perm], axis=0)


def _perm1(x, perm):
    return jnp.concatenate([x[:, p:p + 1, :] for p in perm], axis=1)


def _iota_eye(n):
    ri = jax.lax.broadcasted_iota(jnp.int32, (n, n), 0)
    ci = jax.lax.broadcasted_iota(jnp.int32, (n, n), 1)
    return (ri == ci).astype(jnp.float32)


def _eig_block(pm_ref, adj_ref, adjt_ref, pe_ref):
    adj = adj_ref[...]
    adjt = adjt_ref[...]
    N, _, Ge = adj.shape

    deg = jnp.clip(jnp.sum(adj, axis=1), 1e-6, None)
    dpe = jax.lax.rsqrt(deg)
    eye = _iota_eye(N)
    m = dpe[:, None, :] * adj * dpe[None, :, :]
    mt = dpe[None, :, :] * adjt * dpe[:, None, :]
    lap = eye[:, :, None] * (1.0 + 1e-5) - 0.5 * (m + mt)

    zc = jnp.zeros((N, 1, Ge), dtype=jnp.float32)
    top = jnp.concatenate([lap, zc], axis=1)
    bot = jnp.concatenate(
        [jnp.zeros((1, N, Ge), dtype=jnp.float32),
         jnp.full((1, 1, Ge), _BIG, dtype=jnp.float32)], axis=1)
    A = jnp.concatenate([top, bot], axis=0)

    eyep = _iota_eye(_NP)
    V = eyep[:, :, None] + jnp.zeros((1, 1, Ge), dtype=jnp.float32)
    eyem = eyep[:, :, None]

    def sweep(_, carry):
        A, V, d = carry
        for r, partner in enumerate(_ROUNDS):
            pmj = pm_ref[r][:, :, None]
            apq = jnp.sum(A * jnp.abs(pmj), axis=1)
            ts = jnp.sum(pmj, axis=1)
            dq = _perm0(d, partner)
            apq_safe = jnp.where(apq == 0.0, 1.0, apq)
            tau = (dq - d) / (2.0 * apq_safe)
            sgn = jnp.where(tau > 0.0, 1.0, jnp.where(tau < 0.0, -1.0, ts))
            t = sgn / (jnp.abs(tau) + jnp.sqrt(1.0 + tau * tau))
            t = jnp.where(apq == 0.0, 0.0, t)
            c = jax.lax.rsqrt(1.0 + t * t)
            s = t * c
            S = -s
            d = d - t * apq
            B = c[:, None, :] * A + S[:, None, :] * _perm0(A, partner)
            A = c[None] * B + S[None] * _perm1(B, partner)
            V = c[None] * V + S[None] * _perm1(V, partner)
        return A, V, d

    d0 = jnp.sum(A * eyem, axis=1)
    _, V, ev = jax.lax.fori_loop(0, _NSWEEPS, sweep, (A, V, d0))
    e1 = ev[:, None, :]
    e2 = ev[None, :, :]
    ri = jax.lax.broadcasted_iota(jnp.int32, (_NP, _NP), 0)
    ci = jax.lax.broadcasted_iota(jnp.int32, (_NP, _NP), 1)
    tie = (ri < ci).astype(jnp.float32)
    cnt = (e1 < e2).astype(jnp.float32) + (e1 == e2).astype(jnp.float32) * tie[:, :, None]
    rank = jnp.sum(cnt, axis=0)

    cols = []
    for k in range(_KEIG):
        sel = (rank == float(k)).astype(jnp.float32)
        col = jnp.sum(V * sel[None], axis=1)
        ssum = jnp.sum(col[:N], axis=0, keepdims=True)
        sg = jnp.sign(ssum)
        sg = jnp.where(sg == 0.0, 1.0, sg)
        col = col * sg
        col = jnp.nan_to_num(col, nan=0.0, posinf=1.0, neginf=-1.0)
        cols.append(col[:N, None, :])
    pe_ref[...] = jnp.concatenate(cols, axis=1)


def _mixer_block(x_ref, adj_ref, pe_ref, we_ref, be_ref, W0_ref, b0_ref,
                 W1_ref, b1_ref, g0_ref, bt0_ref, g1_ref, bt1_ref, out_ref):
    x = x_ref[...]
    adj = adj_ref[...]
    pe = pe_ref[...]
    we = we_ref[0, 0]
    be = be_ref[0, 0]
    _, N, _ = x.shape

    mask = (adj > 0).astype(jnp.float32)
    w_t = jax.nn.softplus(adj * we + be) * mask
    eye = jnp.eye(N, dtype=jnp.float32)
    diag_present = jnp.sum(mask * eye[None], axis=-1)
    w_t = w_t + (1.0 - diag_present)[..., None] * eye[None]
    deg = jnp.sum(w_t, axis=-2)
    dinv = jnp.where(deg > 0, jax.lax.rsqrt(deg), 0.0)
    anorm = dinv[:, :, None] * w_t * dinv[:, None, :]

    xpe = jnp.concatenate([x, pe], axis=-1)

    def prop(v):
        return jax.lax.dot_general(anorm, v, (((1,), (1,)), ((0,), (0,))),
                                   preferred_element_type=jnp.float32)

    def ssg(v, W, b):
        h = _ALPHA * v
        vk = v
        for _ in range(_KHOPS):
            vk = prop(vk)
            h = h + (1.0 - _ALPHA) / _KHOPS * vk
        y = jax.lax.dot_general(h, W, (((2,), (1,)), ((), ())),
                                preferred_element_type=jnp.float32)
        return y + b[...][None]

    def ln(v, g, b):
        m = jnp.mean(v, axis=-1, keepdims=True)
        c = v - m
        var = jnp.mean(c * c, axis=-1, keepdims=True)
        return c * jax.lax.rsqrt(var + 1e-5) * g[...][None] + b[...][None]

    h0 = ln(ssg(xpe, W0_ref[...], b0_ref), g0_ref, bt0_ref)
    h1 = ln(ssg(h0, W1_ref[...], b1_ref), g1_ref, bt1_ref)
    out_ref[...] = h0 + h1


def kernel(features, adjacency, we, be, W0, b0, W1, b1, g0, bt0, g1, bt1):
    B, N, T, DM = features.shape
    G = B * T
    x = jnp.transpose(features, (0, 2, 1, 3)).reshape(G, N, DM)
    adj = adjacency.reshape(G, N, N)

    pm_all = np.zeros((len(_ROUNDS), _NP, _NP), dtype=np.float32)
    for r, partner in enumerate(_ROUNDS):
        for i in range(_NP):
            pm_all[r, i, partner[i]] = 1.0 if i < partner[i] else -1.0
    adj_t = jnp.transpose(adj, (1, 2, 0))
    adjt_t = jnp.transpose(adj, (2, 1, 0))
    pe_t = pl.pallas_call(
        _eig_block,
        grid=(G // _GE,),
        in_specs=[
            pl.BlockSpec((len(_ROUNDS), _NP, _NP), lambda i: (0, 0, 0)),
            pl.BlockSpec((N, N, _GE), lambda i: (0, 0, i)),
            pl.BlockSpec((N, N, _GE), lambda i: (0, 0, i)),
        ],
        out_specs=pl.BlockSpec((N, _KEIG, _GE), lambda i: (0, 0, i)),
        out_shape=jax.ShapeDtypeStruct((N, _KEIG, G), jnp.float32),
    )(jnp.asarray(pm_all), adj_t, adjt_t)
    pe = jnp.transpose(pe_t, (2, 0, 1))

    be2 = be.reshape(1, 1)
    b0r, b1r, g0r, bt0r, g1r, bt1r = (
        p.reshape(1, -1) for p in (b0, b1, g0, bt0, g1, bt1))

    full = lambda shape: pl.BlockSpec(shape, lambda i: (0,) * len(shape))
    out = pl.pallas_call(
        _mixer_block,
        grid=(G // _GB,),
        in_specs=[
            pl.BlockSpec((_GB, N, DM), lambda i: (i, 0, 0)),
            pl.BlockSpec((_GB, N, N), lambda i: (i, 0, 0)),
            pl.BlockSpec((_GB, N, _KEIG), lambda i: (i, 0, 0)),
            full((1, 1)),
            full((1, 1)),
            full(W0.shape),
            full((1, DM)),
            full(W1.shape),
            full((1, DM)),
            full((1, DM)),
            full((1, DM)),
            full((1, DM)),
            full((1, DM)),
        ],
        out_specs=pl.BlockSpec((_GB, N, DM), lambda i: (i, 0, 0)),
        out_shape=jax.ShapeDtypeStruct((G, N, DM), jnp.float32),
    )(x, adj, pe, we, be2, W0, b0r, W1, b1r, g0r, bt0r, g1r, bt1r)

    return jnp.transpose(out.reshape(B, T, N, DM), (0, 2, 1, 3))
